# Initial kernel scaffold; baseline (speedup 1.0000x reference)
import jax
import jax.numpy as jnp
from jax.experimental import pallas as pl


def kernel(x_sub, x_hru, ei_ss, ei_hs, ei_sh, params):
    raise NotImplementedError("write your pallas kernel here")


# speedup vs baseline: 8.1194x; 8.1194x over previous
import functools

import jax
import jax.numpy as jnp
from jax import lax
from jax.experimental import pallas as pl
from jax.experimental.pallas import tpu as pltpu
from jax.experimental.pallas import tpu_sc as plsc

N_SUB = 10000
N_HRU = 50000
HID = 128
OUT = 16

NC = 2
NS = 16
NW = NC * NS
K = 128
CW = 16

N_PAD = 10240
ROWS_PT = N_PAD // NS
E_SS_PAD = 163840
E_HS_PAD = 303104


def _pad_edges(ei, e_pad, n_src):
    e = ei.shape[1]
    extra = jnp.arange(e_pad - e, dtype=jnp.int32)
    src = jnp.concatenate([ei[0].astype(jnp.int32), extra % n_src])
    dst = jnp.concatenate([ei[1].astype(jnp.int32),
                           N_SUB + extra % (N_PAD - N_SUB)])
    return src, dst


_SC_PARAMS = pltpu.CompilerParams(use_tc_tiling_on_sc=False)
_SC_MESH = dict(core_axis_name="c", subcore_axis_name="s")


def _make_seg_sum(e_pad):
    e_pw = e_pad // NW
    c = e_pw // K
    assert c % 2 == 0 and c >= 4

    scratch = [
        pltpu.VMEM((K,), jnp.int32), pltpu.VMEM((K,), jnp.int32),
        pltpu.VMEM((K,), jnp.int32), pltpu.VMEM((K,), jnp.int32),
        pltpu.VMEM((K, HID), jnp.float32), pltpu.VMEM((K, HID), jnp.float32),
        pltpu.SemaphoreType.DMA, pltpu.SemaphoreType.DMA,
        pltpu.SemaphoreType.DMA, pltpu.SemaphoreType.DMA,
        pltpu.VMEM_SHARED((N_PAD, HID), jnp.float32),
    ]

    def body(src_h, dst_h, tab_h, out_h,
             srcv0, srcv1, dstv0, dstv1, rows0, rows1,
             g0, g1, s0, s1, acc):
        cid = lax.axis_index("c")
        sid = lax.axis_index("s")
        wid = sid * NC + cid
        base = wid * e_pw
        rbase = sid * ROWS_PT

        srcv = (srcv0, srcv1)
        dstv = (dstv0, dstv1)
        rows = (rows0, rows1)
        gsem = (g0, g1)
        ssem = (s0, s1)

        z16 = jnp.zeros((16,), jnp.float32)

        def zrow(r, _):
            for l in range(HID // 16):
                rows0[r, pl.ds(l * 16, 16)] = z16
            return 0
        lax.fori_loop(0, K, zrow, 0)

        for i in range(ROWS_PT // K):
            pltpu.sync_copy(rows0, acc.at[pl.ds(rbase + i * K, K)])

        plsc.subcore_barrier()

        def load_start(j, b):
            off = base + j * K
            pltpu.sync_copy(src_h.at[pl.ds(off, K)], srcv[b])
            pltpu.sync_copy(dst_h.at[pl.ds(off, K)], dstv[b])
            pltpu.async_copy(tab_h.at[srcv[b]], rows[b], gsem[b])

        def g_wait(b):
            pltpu.make_async_copy(tab_h.at[srcv[b]], rows[b], gsem[b]).wait()

        def s_start(b):
            pltpu.async_copy(rows[b], acc.at[dstv[b]], ssem[b], add=True)

        def s_wait(b):
            pltpu.make_async_copy(rows[b], acc.at[dstv[b]], ssem[b]).wait()

        load_start(0, 0)
        g_wait(0)
        load_start(1, 1)
        s_start(0)

        def pair(jj, _):
            j = 1 + 2 * jj
            g_wait(1)
            s_wait(0)
            load_start(j + 1, 0)
            s_start(1)
            g_wait(0)
            s_wait(1)
            load_start(j + 2, 1)
            s_start(0)
            return 0
        lax.fori_loop(0, (c - 2) // 2, pair, 0)

        g_wait(1)
        s_wait(0)
        s_start(1)
        s_wait(1)

        plsc.subcore_barrier()

        for i in range(ROWS_PT // K):
            pltpu.sync_copy(acc.at[pl.ds(rbase + i * K, K)], rows0)
            pltpu.sync_copy(rows0, out_h.at[cid, pl.ds(rbase + i * K, K)])

    return pl.kernel(
        body,
        out_type=jax.ShapeDtypeStruct((NC, N_PAD, HID), jnp.float32),
        mesh=plsc.VectorSubcoreMesh(**_SC_MESH),
        scratch_types=scratch,
        compiler_params=_SC_PARAMS)


def _make_counts(e_pads):
    chunks = [e // (NW * K) for e in e_pads]
    assert all(c % 2 == 0 and c >= 4 for c in chunks)

    scratch = [
        pltpu.VMEM((K,), jnp.int32), pltpu.VMEM((K,), jnp.int32),
        pltpu.VMEM((K, CW), jnp.float32),
        pltpu.VMEM((K, CW), jnp.float32),
        pltpu.SemaphoreType.DMA, pltpu.SemaphoreType.DMA,
    ] + [pltpu.VMEM_SHARED((N_PAD, CW), jnp.float32) for _ in e_pads]

    def body(*refs):
        n = len(e_pads)
        dst_hs = refs[:n]
        out_hs = refs[n:2 * n]
        dstv0, dstv1, ones, zc, c0, c1 = refs[2 * n:2 * n + 6]
        caccs = refs[2 * n + 6:]

        cid = lax.axis_index("c")
        sid = lax.axis_index("s")
        wid = sid * NC + cid
        rbase = sid * ROWS_PT
        dstv = (dstv0, dstv1)
        csem = (c0, c1)

        z16 = jnp.zeros((16,), jnp.float32)
        onerow = jnp.where(lax.iota(jnp.int32, 16) == 0,
                           1.0, 0.0).astype(jnp.float32)

        def frow(r, _):
            ones[r] = onerow
            zc[r] = z16
            return 0
        lax.fori_loop(0, K, frow, 0)

        for cacc in caccs:
            for i in range(ROWS_PT // K):
                pltpu.sync_copy(zc, cacc.at[pl.ds(rbase + i * K, K)])
        plsc.subcore_barrier()

        for dst_h, cacc, c in zip(dst_hs, caccs, chunks):
            e_pw = c * K
            base = wid * e_pw

            def load(j, b):
                pltpu.sync_copy(dst_h.at[pl.ds(base + j * K, K)], dstv[b])

            def c_start(b):
                pltpu.async_copy(ones, cacc.at[dstv[b]], csem[b], add=True)

            def c_wait(b):
                pltpu.make_async_copy(ones, cacc.at[dstv[b]],
                                      csem[b]).wait()

            load(0, 0)
            c_start(0)
            load(1, 1)
            c_start(1)

            def pair(jj, _):
                j = 2 + 2 * jj
                c_wait(0)
                load(j, 0)
                c_start(0)
                c_wait(1)
                load(j + 1, 1)
                c_start(1)
                return 0
            lax.fori_loop(0, (c - 2) // 2, pair, 0)
            c_wait(0)
            c_wait(1)

        plsc.subcore_barrier()
        for out_h, cacc in zip(out_hs, caccs):
            for i in range(ROWS_PT // K):
                pltpu.sync_copy(cacc.at[pl.ds(rbase + i * K, K)], zc)
                pltpu.sync_copy(zc, out_h.at[cid, pl.ds(rbase + i * K, K)])

    out_type = tuple(jax.ShapeDtypeStruct((NC, N_PAD, CW), jnp.float32)
                     for _ in e_pads)
    return pl.kernel(body, out_type=out_type,
                     mesh=plsc.VectorSubcoreMesh(**_SC_MESH),
                     scratch_types=scratch,
                     compiler_params=_SC_PARAMS)


_BLK = 2000


def _mm_body(x_ref, w_ref, o_ref):
    o_ref[...] = jnp.dot(x_ref[...], w_ref[...],
                         preferred_element_type=jnp.float32)


def _matmul(x, w):
    m, kd = x.shape
    n = w.shape[1]
    return pl.pallas_call(
        _mm_body,
        grid=(m // _BLK,),
        in_specs=[pl.BlockSpec((_BLK, kd), lambda i: (i, 0)),
                  pl.BlockSpec((kd, n), lambda i: (0, 0))],
        out_specs=pl.BlockSpec((_BLK, n), lambda i: (i, 0)),
        out_shape=jax.ShapeDtypeStruct((m, n), jnp.float32),
    )(x, w)


def _layer_body(use_mean, do_relu, final, *refs):
    refs = list(refs)
    x_ref, sp_ref, ap_ref = refs[:3]
    del refs[:3]
    if use_mean:
        css_ref, chs_ref = refs[:2]
        del refs[:2]
    else:
        css_ref = chs_ref = None
    wrss_ref, wlhs_ref, wrhs_ref, bss_ref, bhs_ref, wn_ref = refs[:6]
    del refs[:6]
    if final:
        bn_ref_or_none = refs.pop(0)
    else:
        bn_ref_or_none = None
    outs = refs

    x = x_ref[...]
    s = sp_ref[0] + sp_ref[1]
    a = ap_ref[0] + ap_ref[1]
    if use_mean:
        css = css_ref[0, :, 0:1] + css_ref[1, :, 0:1]
        chs = chs_ref[0, :, 0:1] + chs_ref[1, :, 0:1]
        s = s / jnp.maximum(css, 1.0)
        a = a / jnp.maximum(chs, 1.0)
    out_ss = s + bss_ref[...] + jnp.dot(x, wrss_ref[...],
                                        preferred_element_type=jnp.float32)
    out_hs = (jnp.dot(a, wlhs_ref[...], preferred_element_type=jnp.float32)
              + bhs_ref[...]
              + jnp.dot(x, wrhs_ref[...], preferred_element_type=jnp.float32))
    h = jnp.concatenate([out_ss, out_hs], axis=1)
    if do_relu:
        h = jnp.maximum(h, 0.0)
    if final:
        o = jnp.dot(h, wn_ref[...],
                    preferred_element_type=jnp.float32) + bn_ref_or_none[...]
        m = jnp.max(o, axis=1, keepdims=True)
        e = jnp.exp(o - m)
        outs[0][...] = e / jnp.sum(e, axis=1, keepdims=True)
    else:
        outs[0][...] = h
        outs[1][...] = jnp.dot(h, wn_ref[...],
                               preferred_element_type=jnp.float32)


def _layer(x, sp, ap, css, chs, wrss, wlhs, wrhs, bss, bhs, wn, bn,
           use_mean, do_relu, final):
    d = x.shape[1]
    full = lambda shape: pl.BlockSpec(shape, lambda i: tuple(0 for _ in shape))
    in_specs = [pl.BlockSpec((_BLK, d), lambda i: (i, 0)),
                pl.BlockSpec((NC, _BLK, HID), lambda i: (0, i, 0)),
                pl.BlockSpec((NC, _BLK, HID), lambda i: (0, i, 0))]
    args = [x, sp, ap]
    if use_mean:
        in_specs += [pl.BlockSpec((NC, _BLK, CW), lambda i: (0, i, 0)),
                     pl.BlockSpec((NC, _BLK, CW), lambda i: (0, i, 0))]
        args += [css, chs]
    in_specs += [full(wrss.shape), full(wlhs.shape), full(wrhs.shape),
                 full(bss.shape), full(bhs.shape), full(wn.shape)]
    args += [wrss, wlhs, wrhs, bss, bhs, wn]
    if final:
        in_specs.append(full(bn.shape))
        args.append(bn)
        out_specs = pl.BlockSpec((_BLK, OUT), lambda i: (i, 0))
        out_shape = jax.ShapeDtypeStruct((N_SUB, OUT), jnp.float32)
    else:
        out_specs = [pl.BlockSpec((_BLK, 2 * HID), lambda i: (i, 0)),
                     pl.BlockSpec((_BLK, HID), lambda i: (i, 0))]
        out_shape = [jax.ShapeDtypeStruct((N_SUB, 2 * HID), jnp.float32),
                     jax.ShapeDtypeStruct((N_SUB, HID), jnp.float32)]

    return pl.pallas_call(
        functools.partial(_layer_body, use_mean, do_relu, final),
        grid=(N_SUB // _BLK,),
        in_specs=in_specs,
        out_specs=out_specs,
        out_shape=out_shape,
    )(*args)


def kernel(x_sub, x_hru, ei_ss, ei_hs, ei_sh, params):
    p = params
    src_ss, dst_ss = _pad_edges(ei_ss, E_SS_PAD, N_SUB)
    src_hs, dst_hs = _pad_edges(ei_hs, E_HS_PAD, N_HRU)

    seg_hs = _make_seg_sum(E_HS_PAD)
    seg_ss = _make_seg_sum(E_SS_PAD)
    counts = _make_counts((E_SS_PAD, E_HS_PAD))

    b2 = lambda v: v.reshape(1, -1)

    cnt_ss, cnt_hs = counts(dst_ss, dst_hs)
    ap = seg_hs(src_hs, dst_hs, x_hru)

    p0 = _matmul(x_sub, p['Wl_0_ss'])
    s0p = seg_ss(src_ss, dst_ss, p0)
    sub1, p1 = _layer(x_sub, s0p, ap, None, None,
                      p['Wr_0_ss'], p['Wl_0_hs'], p['Wr_0_hs'],
                      b2(p['bl_0_ss']), b2(p['bl_0_hs']), p['Wl_1_ss'], None,
                      use_mean=False, do_relu=True, final=False)

    s1p = seg_ss(src_ss, dst_ss, p1)
    sub2, p2 = _layer(sub1, s1p, ap, cnt_ss, cnt_hs,
                      p['Wr_1_ss'], p['Wl_1_hs'], p['Wr_1_hs'],
                      b2(p['bl_1_ss']), b2(p['bl_1_hs']), p['Wl_2_ss'], None,
                      use_mean=True, do_relu=True, final=False)

    s2p = seg_ss(src_ss, dst_ss, p2)
    out = _layer(sub2, s2p, ap, cnt_ss, cnt_hs,
                 p['Wr_2_ss'], p['Wl_2_hs'], p['Wr_2_hs'],
                 b2(p['bl_2_ss']), b2(p['bl_2_hs']), p['W_fin'],
                 b2(p['b_fin']),
                 use_mean=True, do_relu=False, final=True)
    return out


# speedup vs baseline: 9.3791x; 1.1552x over previous
import functools

import jax
import jax.numpy as jnp
from jax import lax
from jax.experimental import pallas as pl
from jax.experimental.pallas import tpu as pltpu
from jax.experimental.pallas import tpu_sc as plsc

N_SUB = 10000
N_HRU = 50000
HID = 128
OUT = 16

NC = 2
NS = 16
NW = NC * NS
K = 88
CW = 16

N_PAD = 10240
ROWS_PT = N_PAD // NS
E_SS_PAD = NW * 58 * K
E_HS_PAD = NW * 108 * K


def _spans(total, step):
    out = []
    off = 0
    while off < total:
        out.append((off, min(step, total - off)))
        off += step
    return out


def _pad_edges(ei, e_pad, n_src):
    e = ei.shape[1]
    extra = jnp.arange(e_pad - e, dtype=jnp.int32)
    src = jnp.concatenate([ei[0].astype(jnp.int32), extra % n_src])
    dst = jnp.concatenate([ei[1].astype(jnp.int32),
                           N_SUB + extra % (N_PAD - N_SUB)])
    return src.reshape(e_pad // K, K), dst.reshape(e_pad // K, K)


_SC_PARAMS = pltpu.CompilerParams(use_tc_tiling_on_sc=False)
_SC_MESH = dict(core_axis_name="c", subcore_axis_name="s")


def _make_seg_sum(e_pad):
    c = e_pad // (NW * K)
    assert c % 2 == 0 and c >= 4

    scratch = [
        pltpu.VMEM((c, K), jnp.int32), pltpu.VMEM((c, K), jnp.int32),
        pltpu.VMEM((K, HID), jnp.float32), pltpu.VMEM((K, HID), jnp.float32),
        pltpu.SemaphoreType.DMA, pltpu.SemaphoreType.DMA,
        pltpu.SemaphoreType.DMA, pltpu.SemaphoreType.DMA,
        pltpu.VMEM_SHARED((N_PAD, HID), jnp.float32),
    ]

    def body(src_h, dst_h, tab_h, out_h,
             src_all, dst_all, rows0, rows1, g0, g1, s0, s1, acc):
        cid = lax.axis_index("c")
        sid = lax.axis_index("s")
        wid = sid * NC + cid
        cbase = wid * c
        rbase = sid * ROWS_PT

        rows = (rows0, rows1)
        gsem = (g0, g1)
        ssem = (s0, s1)

        pltpu.sync_copy(src_h.at[pl.ds(cbase, c)], src_all)
        pltpu.sync_copy(dst_h.at[pl.ds(cbase, c)], dst_all)

        z16 = jnp.zeros((16,), jnp.float32)

        def zrow(r, _):
            for l in range(HID // 16):
                rows0[r, pl.ds(l * 16, 16)] = z16
            return 0
        lax.fori_loop(0, K, zrow, 0)

        for off, sz in _spans(ROWS_PT, K):
            pltpu.sync_copy(rows0.at[pl.ds(0, sz)],
                            acc.at[pl.ds(rbase + off, sz)])

        plsc.subcore_barrier()

        def g_start(j, b):
            pltpu.async_copy(tab_h.at[src_all.at[j]], rows[b], gsem[b])

        def g_wait(j, b):
            pltpu.make_async_copy(tab_h.at[src_all.at[j]], rows[b],
                                  gsem[b]).wait()

        def s_start(j, b):
            pltpu.async_copy(rows[b], acc.at[dst_all.at[j]], ssem[b],
                             add=True)

        def s_wait(j, b):
            pltpu.make_async_copy(rows[b], acc.at[dst_all.at[j]],
                                  ssem[b]).wait()

        g_start(0, 0)
        g_wait(0, 0)
        g_start(1, 1)
        s_start(0, 0)

        def pair(jj, _):
            j = 1 + 2 * jj
            g_wait(j, 1)
            s_wait(j - 1, 0)
            g_start(j + 1, 0)
            s_start(j, 1)
            g_wait(j + 1, 0)
            s_wait(j, 1)
            g_start(j + 2, 1)
            s_start(j + 1, 0)
            return 0
        lax.fori_loop(0, (c - 2) // 2, pair, 0)

        g_wait(c - 1, 1)
        s_wait(c - 2, 0)
        s_start(c - 1, 1)
        s_wait(c - 1, 1)

        plsc.subcore_barrier()

        for off, sz in _spans(ROWS_PT, K):
            pltpu.sync_copy(acc.at[pl.ds(rbase + off, sz)],
                            rows0.at[pl.ds(0, sz)])
            pltpu.sync_copy(rows0.at[pl.ds(0, sz)],
                            out_h.at[cid, pl.ds(rbase + off, sz)])

    return pl.kernel(
        body,
        out_type=jax.ShapeDtypeStruct((NC, N_PAD, HID), jnp.float32),
        mesh=plsc.VectorSubcoreMesh(**_SC_MESH),
        scratch_types=scratch,
        compiler_params=_SC_PARAMS)


def _make_counts(e_pads):
    chunks = [e // (NW * K) for e in e_pads]
    assert all(c % 2 == 0 and c >= 4 for c in chunks)

    scratch = [pltpu.VMEM((c, K), jnp.int32) for c in chunks] + [
        pltpu.VMEM((K, CW), jnp.float32),
        pltpu.VMEM((K, CW), jnp.float32),
        pltpu.SemaphoreType.DMA, pltpu.SemaphoreType.DMA,
    ] + [pltpu.VMEM_SHARED((N_PAD, CW), jnp.float32) for _ in e_pads]

    def body(*refs):
        n = len(e_pads)
        dst_hs = refs[:n]
        out_hs = refs[n:2 * n]
        dst_alls = refs[2 * n:3 * n]
        ones, zc, c0, c1 = refs[3 * n:3 * n + 4]
        caccs = refs[3 * n + 4:]

        cid = lax.axis_index("c")
        sid = lax.axis_index("s")
        wid = sid * NC + cid
        rbase = sid * ROWS_PT
        csem = (c0, c1)

        z16 = jnp.zeros((16,), jnp.float32)
        onerow = jnp.where(lax.iota(jnp.int32, 16) == 0,
                           1.0, 0.0).astype(jnp.float32)

        def frow(r, _):
            ones[r] = onerow
            zc[r] = z16
            return 0
        lax.fori_loop(0, K, frow, 0)

        for dst_h, dst_all, c in zip(dst_hs, dst_alls, chunks):
            pltpu.sync_copy(dst_h.at[pl.ds(wid * c, c)], dst_all)
        for cacc in caccs:
            for off, sz in _spans(ROWS_PT, K):
                pltpu.sync_copy(zc.at[pl.ds(0, sz)],
                                cacc.at[pl.ds(rbase + off, sz)])
        plsc.subcore_barrier()

        for dst_all, cacc, c in zip(dst_alls, caccs, chunks):
            def c_start(j, b):
                pltpu.async_copy(ones, cacc.at[dst_all.at[j]], csem[b],
                                 add=True)

            def c_wait(j, b):
                pltpu.make_async_copy(ones, cacc.at[dst_all.at[j]],
                                      csem[b]).wait()

            c_start(0, 0)
            c_start(1, 1)

            def pair(jj, _):
                j = 2 + 2 * jj
                c_wait(j - 2, 0)
                c_start(j, 0)
                c_wait(j - 1, 1)
                c_start(j + 1, 1)
                return 0
            lax.fori_loop(0, (c - 2) // 2, pair, 0)
            c_wait(c - 2, 0)
            c_wait(c - 1, 1)

        plsc.subcore_barrier()
        for out_h, cacc in zip(out_hs, caccs):
            for off, sz in _spans(ROWS_PT, K):
                pltpu.sync_copy(cacc.at[pl.ds(rbase + off, sz)],
                                zc.at[pl.ds(0, sz)])
                pltpu.sync_copy(zc.at[pl.ds(0, sz)],
                                out_h.at[cid, pl.ds(rbase + off, sz)])

    out_type = tuple(jax.ShapeDtypeStruct((NC, N_PAD, CW), jnp.float32)
                     for _ in e_pads)
    return pl.kernel(body, out_type=out_type,
                     mesh=plsc.VectorSubcoreMesh(**_SC_MESH),
                     scratch_types=scratch,
                     compiler_params=_SC_PARAMS)


_BLK = 2000


def _mm_body(x_ref, w_ref, o_ref):
    o_ref[...] = jnp.dot(x_ref[...], w_ref[...],
                         preferred_element_type=jnp.float32, precision=lax.Precision.HIGHEST)


def _matmul(x, w):
    m, kd = x.shape
    n = w.shape[1]
    return pl.pallas_call(
        _mm_body,
        grid=(m // _BLK,),
        in_specs=[pl.BlockSpec((_BLK, kd), lambda i: (i, 0)),
                  pl.BlockSpec((kd, n), lambda i: (0, 0))],
        out_specs=pl.BlockSpec((_BLK, n), lambda i: (i, 0)),
        out_shape=jax.ShapeDtypeStruct((m, n), jnp.float32),
    )(x, w)


def _layer_body(use_mean, do_relu, final, *refs):
    refs = list(refs)
    x_ref, sp_ref, ap_ref = refs[:3]
    del refs[:3]
    if use_mean:
        css_ref, chs_ref = refs[:2]
        del refs[:2]
    else:
        css_ref = chs_ref = None
    wrss_ref, wlhs_ref, wrhs_ref, bss_ref, bhs_ref, wn_ref = refs[:6]
    del refs[:6]
    if final:
        bn_ref_or_none = refs.pop(0)
    else:
        bn_ref_or_none = None
    outs = refs

    x = x_ref[...]
    s = sp_ref[0] + sp_ref[1]
    a = ap_ref[0] + ap_ref[1]
    if use_mean:
        css = css_ref[0, :, 0:1] + css_ref[1, :, 0:1]
        chs = chs_ref[0, :, 0:1] + chs_ref[1, :, 0:1]
        s = s / jnp.maximum(css, 1.0)
        a = a / jnp.maximum(chs, 1.0)
    out_ss = s + bss_ref[...] + jnp.dot(x, wrss_ref[...],
                                        preferred_element_type=jnp.float32, precision=lax.Precision.HIGHEST)
    out_hs = (jnp.dot(a, wlhs_ref[...], preferred_element_type=jnp.float32, precision=lax.Precision.HIGHEST)
              + bhs_ref[...]
              + jnp.dot(x, wrhs_ref[...], preferred_element_type=jnp.float32, precision=lax.Precision.HIGHEST))
    h = jnp.concatenate([out_ss, out_hs], axis=1)
    if do_relu:
        h = jnp.maximum(h, 0.0)
    if final:
        o = jnp.dot(h, wn_ref[...],
                    preferred_element_type=jnp.float32, precision=lax.Precision.HIGHEST) + bn_ref_or_none[...]
        m = jnp.max(o, axis=1, keepdims=True)
        e = jnp.exp(o - m)
        outs[0][...] = e / jnp.sum(e, axis=1, keepdims=True)
    else:
        outs[0][...] = h
        outs[1][...] = jnp.dot(h, wn_ref[...],
                               preferred_element_type=jnp.float32, precision=lax.Precision.HIGHEST)


def _layer(x, sp, ap, css, chs, wrss, wlhs, wrhs, bss, bhs, wn, bn,
           use_mean, do_relu, final):
    d = x.shape[1]
    full = lambda shape: pl.BlockSpec(shape, lambda i: tuple(0 for _ in shape))
    in_specs = [pl.BlockSpec((_BLK, d), lambda i: (i, 0)),
                pl.BlockSpec((NC, _BLK, HID), lambda i: (0, i, 0)),
                pl.BlockSpec((NC, _BLK, HID), lambda i: (0, i, 0))]
    args = [x, sp, ap]
    if use_mean:
        in_specs += [pl.BlockSpec((NC, _BLK, CW), lambda i: (0, i, 0)),
                     pl.BlockSpec((NC, _BLK, CW), lambda i: (0, i, 0))]
        args += [css, chs]
    in_specs += [full(wrss.shape), full(wlhs.shape), full(wrhs.shape),
                 full(bss.shape), full(bhs.shape), full(wn.shape)]
    args += [wrss, wlhs, wrhs, bss, bhs, wn]
    if final:
        in_specs.append(full(bn.shape))
        args.append(bn)
        out_specs = pl.BlockSpec((_BLK, OUT), lambda i: (i, 0))
        out_shape = jax.ShapeDtypeStruct((N_SUB, OUT), jnp.float32)
    else:
        out_specs = [pl.BlockSpec((_BLK, 2 * HID), lambda i: (i, 0)),
                     pl.BlockSpec((_BLK, HID), lambda i: (i, 0))]
        out_shape = [jax.ShapeDtypeStruct((N_SUB, 2 * HID), jnp.float32),
                     jax.ShapeDtypeStruct((N_SUB, HID), jnp.float32)]

    return pl.pallas_call(
        functools.partial(_layer_body, use_mean, do_relu, final),
        grid=(N_SUB // _BLK,),
        in_specs=in_specs,
        out_specs=out_specs,
        out_shape=out_shape,
    )(*args)


def kernel(x_sub, x_hru, ei_ss, ei_hs, ei_sh, params):
    p = params
    src_ss, dst_ss = _pad_edges(ei_ss, E_SS_PAD, N_SUB)
    src_hs, dst_hs = _pad_edges(ei_hs, E_HS_PAD, N_HRU)

    seg_hs = _make_seg_sum(E_HS_PAD)
    seg_ss = _make_seg_sum(E_SS_PAD)
    counts = _make_counts((E_SS_PAD, E_HS_PAD))

    b2 = lambda v: v.reshape(1, -1)

    cnt_ss, cnt_hs = counts(dst_ss, dst_hs)
    ap = seg_hs(src_hs, dst_hs, x_hru)

    p0 = _matmul(x_sub, p['Wl_0_ss'])
    s0p = seg_ss(src_ss, dst_ss, p0)
    sub1, p1 = _layer(x_sub, s0p, ap, None, None,
                      p['Wr_0_ss'], p['Wl_0_hs'], p['Wr_0_hs'],
                      b2(p['bl_0_ss']), b2(p['bl_0_hs']), p['Wl_1_ss'], None,
                      use_mean=False, do_relu=True, final=False)

    s1p = seg_ss(src_ss, dst_ss, p1)
    sub2, p2 = _layer(sub1, s1p, ap, cnt_ss, cnt_hs,
                      p['Wr_1_ss'], p['Wl_1_hs'], p['Wr_1_hs'],
                      b2(p['bl_1_ss']), b2(p['bl_1_hs']), p['Wl_2_ss'], None,
                      use_mean=True, do_relu=True, final=False)

    s2p = seg_ss(src_ss, dst_ss, p2)
    out = _layer(sub2, s2p, ap, cnt_ss, cnt_hs,
                 p['Wr_2_ss'], p['Wl_2_hs'], p['Wr_2_hs'],
                 b2(p['bl_2_ss']), b2(p['bl_2_hs']), p['W_fin'],
                 b2(p['b_fin']),
                 use_mean=True, do_relu=False, final=True)
    return out


# speedup vs baseline: 10.4328x; 1.1123x over previous
import functools

import jax
import jax.numpy as jnp
from jax import lax
from jax.experimental import pallas as pl
from jax.experimental.pallas import tpu as pltpu
from jax.experimental.pallas import tpu_sc as plsc

N_SUB = 10000
N_HRU = 50000
HID = 128
OUT = 16

NC = 2
NS = 16
NW = NC * NS
K = 56
CW = 16

N_PAD = 10240
ROWS_PT = N_PAD // NS
E_SS_PAD = NW * 92 * K
E_HS_PAD = NW * 172 * K


def _spans(total, step):
    out = []
    off = 0
    while off < total:
        out.append((off, min(step, total - off)))
        off += step
    return out


def _pad_edges(ei, e_pad, n_src):
    e = ei.shape[1]
    extra = jnp.arange(e_pad - e, dtype=jnp.int32)
    src = jnp.concatenate([ei[0].astype(jnp.int32), extra % n_src])
    dst = jnp.concatenate([ei[1].astype(jnp.int32),
                           N_SUB + extra % (N_PAD - N_SUB)])
    return src.reshape(e_pad // K, K), dst.reshape(e_pad // K, K)


_SC_PARAMS = pltpu.CompilerParams(use_tc_tiling_on_sc=False)
_SC_MESH = dict(core_axis_name="c", subcore_axis_name="s")


def _make_seg_sum(e_pad):
    c = e_pad // (NW * K)
    assert c % 4 == 0 and c >= 8

    scratch = [
        pltpu.VMEM((c, K), jnp.int32), pltpu.VMEM((c, K), jnp.int32),
    ] + [pltpu.VMEM((K, HID), jnp.float32) for _ in range(4)] + [
        pltpu.SemaphoreType.DMA for _ in range(8)
    ] + [pltpu.VMEM_SHARED((N_PAD, HID), jnp.float32)]

    def body(src_h, dst_h, tab_h, out_h,
             src_all, dst_all, r0, r1, r2, r3,
             g0, g1, g2, g3, s0, s1, s2, s3, acc):
        cid = lax.axis_index("c")
        sid = lax.axis_index("s")
        wid = sid * NC + cid
        cbase = wid * c
        rbase = sid * ROWS_PT

        rows = (r0, r1, r2, r3)
        gsem = (g0, g1, g2, g3)
        ssem = (s0, s1, s2, s3)
        rows0 = r0

        pltpu.sync_copy(src_h.at[pl.ds(cbase, c)], src_all)
        pltpu.sync_copy(dst_h.at[pl.ds(cbase, c)], dst_all)

        z16 = jnp.zeros((16,), jnp.float32)

        def zrow(r, _):
            for l in range(HID // 16):
                rows0[r, pl.ds(l * 16, 16)] = z16
            return 0
        lax.fori_loop(0, K, zrow, 0)

        for off, sz in _spans(ROWS_PT, K):
            pltpu.sync_copy(rows0.at[pl.ds(0, sz)],
                            acc.at[pl.ds(rbase + off, sz)])

        plsc.subcore_barrier()

        def g_start(j, b):
            pltpu.async_copy(tab_h.at[src_all.at[j]], rows[b], gsem[b])

        def g_wait(j, b):
            pltpu.make_async_copy(tab_h.at[src_all.at[j]], rows[b],
                                  gsem[b]).wait()

        def s_start(j, b):
            pltpu.async_copy(rows[b], acc.at[dst_all.at[j]], ssem[b],
                             add=True)

        def s_wait(j, b):
            pltpu.make_async_copy(rows[b], acc.at[dst_all.at[j]],
                                  ssem[b]).wait()

        g_start(0, 0)
        g_start(1, 1)
        g_wait(0, 0)
        s_start(0, 0)
        g_start(2, 2)
        g_wait(1, 1)
        s_start(1, 1)
        g_start(3, 3)

        def quad(g, _):
            j0 = 2 + 4 * g
            for t, b in enumerate((2, 3, 0, 1)):
                j = j0 + t
                g_wait(j, b)
                s_start(j, b)
                s_wait(j - 2, (b + 2) % 4)
                g_start(j + 2, (b + 2) % 4)
            return 0
        lax.fori_loop(0, (c - 4) // 4, quad, 0)

        g_wait(c - 2, 2)
        s_start(c - 2, 2)
        s_wait(c - 4, 0)
        g_wait(c - 1, 3)
        s_start(c - 1, 3)
        s_wait(c - 3, 1)
        s_wait(c - 2, 2)
        s_wait(c - 1, 3)

        plsc.subcore_barrier()

        for off, sz in _spans(ROWS_PT, K):
            pltpu.sync_copy(acc.at[pl.ds(rbase + off, sz)],
                            rows0.at[pl.ds(0, sz)])
            pltpu.sync_copy(rows0.at[pl.ds(0, sz)],
                            out_h.at[cid, pl.ds(rbase + off, sz)])

    return pl.kernel(
        body,
        out_type=jax.ShapeDtypeStruct((NC, N_PAD, HID), jnp.float32),
        mesh=plsc.VectorSubcoreMesh(**_SC_MESH),
        scratch_types=scratch,
        compiler_params=_SC_PARAMS)


def _make_counts(e_pads):
    chunks = [e // (NW * K) for e in e_pads]
    assert all(c % 2 == 0 and c >= 4 for c in chunks)

    scratch = [pltpu.VMEM((c, K), jnp.int32) for c in chunks] + [
        pltpu.VMEM((K, CW), jnp.float32),
        pltpu.VMEM((K, CW), jnp.float32),
        pltpu.SemaphoreType.DMA, pltpu.SemaphoreType.DMA,
    ] + [pltpu.VMEM_SHARED((N_PAD, CW), jnp.float32) for _ in e_pads]

    def body(*refs):
        n = len(e_pads)
        dst_hs = refs[:n]
        out_hs = refs[n:2 * n]
        dst_alls = refs[2 * n:3 * n]
        ones, zc, c0, c1 = refs[3 * n:3 * n + 4]
        caccs = refs[3 * n + 4:]

        cid = lax.axis_index("c")
        sid = lax.axis_index("s")
        wid = sid * NC + cid
        rbase = sid * ROWS_PT
        csem = (c0, c1)

        z16 = jnp.zeros((16,), jnp.float32)
        onerow = jnp.where(lax.iota(jnp.int32, 16) == 0,
                           1.0, 0.0).astype(jnp.float32)

        def frow(r, _):
            ones[r] = onerow
            zc[r] = z16
            return 0
        lax.fori_loop(0, K, frow, 0)

        for dst_h, dst_all, c in zip(dst_hs, dst_alls, chunks):
            pltpu.sync_copy(dst_h.at[pl.ds(wid * c, c)], dst_all)
        for cacc in caccs:
            for off, sz in _spans(ROWS_PT, K):
                pltpu.sync_copy(zc.at[pl.ds(0, sz)],
                                cacc.at[pl.ds(rbase + off, sz)])
        plsc.subcore_barrier()

        for dst_all, cacc, c in zip(dst_alls, caccs, chunks):
            def c_start(j, b):
                pltpu.async_copy(ones, cacc.at[dst_all.at[j]], csem[b],
                                 add=True)

            def c_wait(j, b):
                pltpu.make_async_copy(ones, cacc.at[dst_all.at[j]],
                                      csem[b]).wait()

            c_start(0, 0)
            c_start(1, 1)

            def pair(jj, _):
                j = 2 + 2 * jj
                c_wait(j - 2, 0)
                c_start(j, 0)
                c_wait(j - 1, 1)
                c_start(j + 1, 1)
                return 0
            lax.fori_loop(0, (c - 2) // 2, pair, 0)
            c_wait(c - 2, 0)
            c_wait(c - 1, 1)

        plsc.subcore_barrier()
        for out_h, cacc in zip(out_hs, caccs):
            for off, sz in _spans(ROWS_PT, K):
                pltpu.sync_copy(cacc.at[pl.ds(rbase + off, sz)],
                                zc.at[pl.ds(0, sz)])
                pltpu.sync_copy(zc.at[pl.ds(0, sz)],
                                out_h.at[cid, pl.ds(rbase + off, sz)])

    out_type = tuple(jax.ShapeDtypeStruct((NC, N_PAD, CW), jnp.float32)
                     for _ in e_pads)
    return pl.kernel(body, out_type=out_type,
                     mesh=plsc.VectorSubcoreMesh(**_SC_MESH),
                     scratch_types=scratch,
                     compiler_params=_SC_PARAMS)


_BLK = 2000


def _mm_body(x_ref, w_ref, o_ref):
    o_ref[...] = jnp.dot(x_ref[...], w_ref[...],
                         preferred_element_type=jnp.float32, precision=lax.Precision.HIGHEST)


def _matmul(x, w):
    m, kd = x.shape
    n = w.shape[1]
    return pl.pallas_call(
        _mm_body,
        grid=(m // _BLK,),
        in_specs=[pl.BlockSpec((_BLK, kd), lambda i: (i, 0)),
                  pl.BlockSpec((kd, n), lambda i: (0, 0))],
        out_specs=pl.BlockSpec((_BLK, n), lambda i: (i, 0)),
        out_shape=jax.ShapeDtypeStruct((m, n), jnp.float32),
    )(x, w)


def _layer_body(use_mean, do_relu, final, *refs):
    refs = list(refs)
    x_ref, sp_ref, ap_ref = refs[:3]
    del refs[:3]
    if use_mean:
        css_ref, chs_ref = refs[:2]
        del refs[:2]
    else:
        css_ref = chs_ref = None
    wrss_ref, wlhs_ref, wrhs_ref, bss_ref, bhs_ref, wn_ref = refs[:6]
    del refs[:6]
    if final:
        bn_ref_or_none = refs.pop(0)
    else:
        bn_ref_or_none = None
    outs = refs

    x = x_ref[...]
    s = sp_ref[0] + sp_ref[1]
    a = ap_ref[0] + ap_ref[1]
    if use_mean:
        css = css_ref[0, :, 0:1] + css_ref[1, :, 0:1]
        chs = chs_ref[0, :, 0:1] + chs_ref[1, :, 0:1]
        s = s / jnp.maximum(css, 1.0)
        a = a / jnp.maximum(chs, 1.0)
    out_ss = s + bss_ref[...] + jnp.dot(x, wrss_ref[...],
                                        preferred_element_type=jnp.float32, precision=lax.Precision.HIGHEST)
    out_hs = (jnp.dot(a, wlhs_ref[...], preferred_element_type=jnp.float32, precision=lax.Precision.HIGHEST)
              + bhs_ref[...]
              + jnp.dot(x, wrhs_ref[...], preferred_element_type=jnp.float32, precision=lax.Precision.HIGHEST))
    h = jnp.concatenate([out_ss, out_hs], axis=1)
    if do_relu:
        h = jnp.maximum(h, 0.0)
    if final:
        o = jnp.dot(h, wn_ref[...],
                    preferred_element_type=jnp.float32, precision=lax.Precision.HIGHEST) + bn_ref_or_none[...]
        m = jnp.max(o, axis=1, keepdims=True)
        e = jnp.exp(o - m)
        outs[0][...] = e / jnp.sum(e, axis=1, keepdims=True)
    else:
        outs[0][...] = h
        outs[1][...] = jnp.dot(h, wn_ref[...],
                               preferred_element_type=jnp.float32, precision=lax.Precision.HIGHEST)


def _layer(x, sp, ap, css, chs, wrss, wlhs, wrhs, bss, bhs, wn, bn,
           use_mean, do_relu, final):
    d = x.shape[1]
    full = lambda shape: pl.BlockSpec(shape, lambda i: tuple(0 for _ in shape))
    in_specs = [pl.BlockSpec((_BLK, d), lambda i: (i, 0)),
                pl.BlockSpec((NC, _BLK, HID), lambda i: (0, i, 0)),
                pl.BlockSpec((NC, _BLK, HID), lambda i: (0, i, 0))]
    args = [x, sp, ap]
    if use_mean:
        in_specs += [pl.BlockSpec((NC, _BLK, CW), lambda i: (0, i, 0)),
                     pl.BlockSpec((NC, _BLK, CW), lambda i: (0, i, 0))]
        args += [css, chs]
    in_specs += [full(wrss.shape), full(wlhs.shape), full(wrhs.shape),
                 full(bss.shape), full(bhs.shape), full(wn.shape)]
    args += [wrss, wlhs, wrhs, bss, bhs, wn]
    if final:
        in_specs.append(full(bn.shape))
        args.append(bn)
        out_specs = pl.BlockSpec((_BLK, OUT), lambda i: (i, 0))
        out_shape = jax.ShapeDtypeStruct((N_SUB, OUT), jnp.float32)
    else:
        out_specs = [pl.BlockSpec((_BLK, 2 * HID), lambda i: (i, 0)),
                     pl.BlockSpec((_BLK, HID), lambda i: (i, 0))]
        out_shape = [jax.ShapeDtypeStruct((N_SUB, 2 * HID), jnp.float32),
                     jax.ShapeDtypeStruct((N_SUB, HID), jnp.float32)]

    return pl.pallas_call(
        functools.partial(_layer_body, use_mean, do_relu, final),
        grid=(N_SUB // _BLK,),
        in_specs=in_specs,
        out_specs=out_specs,
        out_shape=out_shape,
    )(*args)


def kernel(x_sub, x_hru, ei_ss, ei_hs, ei_sh, params):
    p = params
    src_ss, dst_ss = _pad_edges(ei_ss, E_SS_PAD, N_SUB)
    src_hs, dst_hs = _pad_edges(ei_hs, E_HS_PAD, N_HRU)

    seg_hs = _make_seg_sum(E_HS_PAD)
    seg_ss = _make_seg_sum(E_SS_PAD)
    counts = _make_counts((E_SS_PAD, E_HS_PAD))

    b2 = lambda v: v.reshape(1, -1)

    cnt_ss, cnt_hs = counts(dst_ss, dst_hs)
    ap = seg_hs(src_hs, dst_hs, x_hru)

    p0 = _matmul(x_sub, p['Wl_0_ss'])
    s0p = seg_ss(src_ss, dst_ss, p0)
    sub1, p1 = _layer(x_sub, s0p, ap, None, None,
                      p['Wr_0_ss'], p['Wl_0_hs'], p['Wr_0_hs'],
                      b2(p['bl_0_ss']), b2(p['bl_0_hs']), p['Wl_1_ss'], None,
                      use_mean=False, do_relu=True, final=False)

    s1p = seg_ss(src_ss, dst_ss, p1)
    sub2, p2 = _layer(sub1, s1p, ap, cnt_ss, cnt_hs,
                      p['Wr_1_ss'], p['Wl_1_hs'], p['Wr_1_hs'],
                      b2(p['bl_1_ss']), b2(p['bl_1_hs']), p['Wl_2_ss'], None,
                      use_mean=True, do_relu=True, final=False)

    s2p = seg_ss(src_ss, dst_ss, p2)
    out = _layer(sub2, s2p, ap, cnt_ss, cnt_hs,
                 p['Wr_2_ss'], p['Wl_2_hs'], p['Wr_2_hs'],
                 b2(p['bl_2_ss']), b2(p['bl_2_hs']), p['W_fin'],
                 b2(p['b_fin']),
                 use_mean=True, do_relu=False, final=True)
    return out


# speedup vs baseline: 11.8292x; 1.1338x over previous
import functools

import jax
import jax.numpy as jnp
from jax import lax
from jax.experimental import pallas as pl
from jax.experimental.pallas import tpu as pltpu
from jax.experimental.pallas import tpu_sc as plsc

N_SUB = 10000
N_HRU = 50000
HID = 128
OUT = 16

NC = 2
NS = 16
NW = NC * NS
K = 56
CW = 16

N_PAD = 10240
ROWS_PT = N_PAD // NS
E_SS_PAD = NW * 92 * K
E_HS_PAD = NW * 172 * K


def _spans(total, step):
    out = []
    off = 0
    while off < total:
        out.append((off, min(step, total - off)))
        off += step
    return out


def _pad_edges(ei, e_pad, n_src):
    e = ei.shape[1]
    extra = jnp.arange(e_pad - e, dtype=jnp.int32)
    src = jnp.concatenate([ei[0].astype(jnp.int32), extra % n_src])
    dst = jnp.concatenate([ei[1].astype(jnp.int32),
                           N_SUB + extra % (N_PAD - N_SUB)])
    return src.reshape(e_pad // K, K), dst.reshape(e_pad // K, K)


_SC_PARAMS = pltpu.CompilerParams(use_tc_tiling_on_sc=False)
_SC_MESH = dict(core_axis_name="c", subcore_axis_name="s")


def _make_seg_sum(e_pad):
    c = e_pad // (NW * K)
    assert c % 4 == 0 and c >= 8

    scratch = [
        pltpu.VMEM((c, K), jnp.int32), pltpu.VMEM((c, K), jnp.int32),
    ] + [pltpu.VMEM((K, HID), jnp.float32) for _ in range(4)] + [
        pltpu.SemaphoreType.DMA for _ in range(8)
    ] + [pltpu.VMEM_SHARED((N_PAD, HID), jnp.float32)]

    def body(src_h, dst_h, tab_h, out_h,
             src_all, dst_all, r0, r1, r2, r3,
             g0, g1, g2, g3, s0, s1, s2, s3, acc):
        cid = lax.axis_index("c")
        sid = lax.axis_index("s")
        wid = sid * NC + cid
        cbase = wid * c
        rbase = sid * ROWS_PT

        rows = (r0, r1, r2, r3)
        gsem = (g0, g1, g2, g3)
        ssem = (s0, s1, s2, s3)
        rows0 = r0

        pltpu.sync_copy(src_h.at[pl.ds(cbase, c)], src_all)
        pltpu.sync_copy(dst_h.at[pl.ds(cbase, c)], dst_all)

        z16 = jnp.zeros((16,), jnp.float32)

        def zrow(r, _):
            for l in range(HID // 16):
                rows0[r, pl.ds(l * 16, 16)] = z16
            return 0
        lax.fori_loop(0, K, zrow, 0)

        for off, sz in _spans(ROWS_PT, K):
            pltpu.sync_copy(rows0.at[pl.ds(0, sz)],
                            acc.at[pl.ds(rbase + off, sz)])

        plsc.subcore_barrier()

        def g_start(j, b):
            pltpu.async_copy(tab_h.at[src_all.at[j]], rows[b], gsem[b])

        def g_wait(j, b):
            pltpu.make_async_copy(tab_h.at[src_all.at[j]], rows[b],
                                  gsem[b]).wait()

        def s_start(j, b):
            pltpu.async_copy(rows[b], acc.at[dst_all.at[j]], ssem[b],
                             add=True)

        def s_wait(j, b):
            pltpu.make_async_copy(rows[b], acc.at[dst_all.at[j]],
                                  ssem[b]).wait()

        g_start(0, 0)
        g_start(1, 1)
        g_wait(0, 0)
        s_start(0, 0)
        g_start(2, 2)
        g_wait(1, 1)
        s_start(1, 1)
        g_start(3, 3)

        def quad(g, _):
            j0 = 2 + 4 * g
            for t, b in enumerate((2, 3, 0, 1)):
                j = j0 + t
                g_wait(j, b)
                s_start(j, b)
                s_wait(j - 2, (b + 2) % 4)
                g_start(j + 2, (b + 2) % 4)
            return 0
        lax.fori_loop(0, (c - 4) // 4, quad, 0)

        g_wait(c - 2, 2)
        s_start(c - 2, 2)
        s_wait(c - 4, 0)
        g_wait(c - 1, 3)
        s_start(c - 1, 3)
        s_wait(c - 3, 1)
        s_wait(c - 2, 2)
        s_wait(c - 1, 3)

        plsc.subcore_barrier()

        for off, sz in _spans(ROWS_PT, K):
            pltpu.sync_copy(acc.at[pl.ds(rbase + off, sz)],
                            rows0.at[pl.ds(0, sz)])
            pltpu.sync_copy(rows0.at[pl.ds(0, sz)],
                            out_h.at[cid, pl.ds(rbase + off, sz)])

    return pl.kernel(
        body,
        out_type=jax.ShapeDtypeStruct((NC, N_PAD, HID), jnp.float32),
        mesh=plsc.VectorSubcoreMesh(**_SC_MESH),
        scratch_types=scratch,
        compiler_params=_SC_PARAMS)


def _make_counts(e_pads):
    chunks = [e // (NW * K) for e in e_pads]
    assert all(c % 2 == 0 and c >= 4 for c in chunks)

    scratch = [pltpu.VMEM((c, K), jnp.int32) for c in chunks] + [
        pltpu.VMEM((K, CW), jnp.float32),
        pltpu.VMEM((K, CW), jnp.float32),
        pltpu.SemaphoreType.DMA, pltpu.SemaphoreType.DMA,
    ] + [pltpu.VMEM_SHARED((N_PAD, CW), jnp.float32) for _ in e_pads]

    def body(*refs):
        n = len(e_pads)
        dst_hs = refs[:n]
        out_hs = refs[n:2 * n]
        dst_alls = refs[2 * n:3 * n]
        ones, zc, c0, c1 = refs[3 * n:3 * n + 4]
        caccs = refs[3 * n + 4:]

        cid = lax.axis_index("c")
        sid = lax.axis_index("s")
        wid = sid * NC + cid
        rbase = sid * ROWS_PT
        csem = (c0, c1)

        z16 = jnp.zeros((16,), jnp.float32)
        onerow = jnp.where(lax.iota(jnp.int32, 16) == 0,
                           1.0, 0.0).astype(jnp.float32)

        def frow(r, _):
            ones[r] = onerow
            zc[r] = z16
            return 0
        lax.fori_loop(0, K, frow, 0)

        for dst_h, dst_all, c in zip(dst_hs, dst_alls, chunks):
            pltpu.sync_copy(dst_h.at[pl.ds(wid * c, c)], dst_all)
        for cacc in caccs:
            for off, sz in _spans(ROWS_PT, K):
                pltpu.sync_copy(zc.at[pl.ds(0, sz)],
                                cacc.at[pl.ds(rbase + off, sz)])
        plsc.subcore_barrier()

        for dst_all, cacc, c in zip(dst_alls, caccs, chunks):
            def c_start(j, b):
                pltpu.async_copy(ones, cacc.at[dst_all.at[j]], csem[b],
                                 add=True)

            def c_wait(j, b):
                pltpu.make_async_copy(ones, cacc.at[dst_all.at[j]],
                                      csem[b]).wait()

            c_start(0, 0)
            c_start(1, 1)

            def pair(jj, _):
                j = 2 + 2 * jj
                c_wait(j - 2, 0)
                c_start(j, 0)
                c_wait(j - 1, 1)
                c_start(j + 1, 1)
                return 0
            lax.fori_loop(0, (c - 2) // 2, pair, 0)
            c_wait(c - 2, 0)
            c_wait(c - 1, 1)

        plsc.subcore_barrier()
        for out_h, cacc in zip(out_hs, caccs):
            for off, sz in _spans(ROWS_PT, K):
                pltpu.sync_copy(cacc.at[pl.ds(rbase + off, sz)],
                                zc.at[pl.ds(0, sz)])
                pltpu.sync_copy(zc.at[pl.ds(0, sz)],
                                out_h.at[cid, pl.ds(rbase + off, sz)])

    out_type = tuple(jax.ShapeDtypeStruct((NC, N_PAD, CW), jnp.float32)
                     for _ in e_pads)
    return pl.kernel(body, out_type=out_type,
                     mesh=plsc.VectorSubcoreMesh(**_SC_MESH),
                     scratch_types=scratch,
                     compiler_params=_SC_PARAMS)


_BLK = 2000


def _dot(a, b):
    return jnp.dot(a, b, preferred_element_type=jnp.float32)


def _dot_hi(a, b):
    return jnp.dot(a, b, preferred_element_type=jnp.float32,
                   precision=lax.Precision.HIGHEST)


def _layer_body(use_mean, do_relu, final, proj_ss, *refs):
    refs = list(refs)
    x_ref, sp_ref, ap_ref = refs[:3]
    del refs[:3]
    if use_mean:
        css_ref, chs_ref = refs[:2]
        del refs[:2]
    else:
        css_ref = chs_ref = None
    if proj_ss:
        wlss_ref = refs.pop(0)
    else:
        wlss_ref = None
    wrss_ref, wlhs_ref, wrhs_ref, bss_ref, bhs_ref, wn_ref = refs[:6]
    del refs[:6]
    if final:
        bn_ref_or_none = refs.pop(0)
    else:
        bn_ref_or_none = None
    outs = refs

    x = x_ref[...]
    s = sp_ref[0] + sp_ref[1]
    a = ap_ref[0] + ap_ref[1]
    if use_mean:
        css = css_ref[0, :, 0:1] + css_ref[1, :, 0:1]
        chs = chs_ref[0, :, 0:1] + chs_ref[1, :, 0:1]
        s = s / jnp.maximum(css, 1.0)
        a = a / jnp.maximum(chs, 1.0)
    if proj_ss:
        s = _dot(s, wlss_ref[...])
    out_ss = s + bss_ref[...] + _dot(x, wrss_ref[...])
    out_hs = _dot(a, wlhs_ref[...]) + bhs_ref[...] + _dot(x, wrhs_ref[...])
    h = jnp.concatenate([out_ss, out_hs], axis=1)
    if do_relu:
        h = jnp.maximum(h, 0.0)
    if final:
        o = _dot(h, wn_ref[...]) + bn_ref_or_none[...]
        m = jnp.max(o, axis=1, keepdims=True)
        e = jnp.exp(o - m)
        outs[0][...] = e / jnp.sum(e, axis=1, keepdims=True)
    else:
        outs[0][...] = h
        outs[1][...] = _dot_hi(h, wn_ref[...])


def _layer(x, sp, ap, css, chs, wrss, wlhs, wrhs, bss, bhs, wn, bn,
           use_mean, do_relu, final, wlss=None):
    d = x.shape[1]
    full = lambda shape: pl.BlockSpec(shape, lambda i: tuple(0 for _ in shape))
    in_specs = [pl.BlockSpec((_BLK, d), lambda i: (i, 0)),
                pl.BlockSpec((NC, _BLK, HID), lambda i: (0, i, 0)),
                pl.BlockSpec((NC, _BLK, HID), lambda i: (0, i, 0))]
    args = [x, sp, ap]
    if use_mean:
        in_specs += [pl.BlockSpec((NC, _BLK, CW), lambda i: (0, i, 0)),
                     pl.BlockSpec((NC, _BLK, CW), lambda i: (0, i, 0))]
        args += [css, chs]
    if wlss is not None:
        in_specs.append(full(wlss.shape))
        args.append(wlss)
    in_specs += [full(wrss.shape), full(wlhs.shape), full(wrhs.shape),
                 full(bss.shape), full(bhs.shape), full(wn.shape)]
    args += [wrss, wlhs, wrhs, bss, bhs, wn]
    if final:
        in_specs.append(full(bn.shape))
        args.append(bn)
        out_specs = pl.BlockSpec((_BLK, OUT), lambda i: (i, 0))
        out_shape = jax.ShapeDtypeStruct((N_SUB, OUT), jnp.float32)
    else:
        out_specs = [pl.BlockSpec((_BLK, 2 * HID), lambda i: (i, 0)),
                     pl.BlockSpec((_BLK, HID), lambda i: (i, 0))]
        out_shape = [jax.ShapeDtypeStruct((N_SUB, 2 * HID), jnp.float32),
                     jax.ShapeDtypeStruct((N_SUB, HID), jnp.float32)]

    return pl.pallas_call(
        functools.partial(_layer_body, use_mean, do_relu, final,
                          wlss is not None),
        grid=(N_SUB // _BLK,),
        in_specs=in_specs,
        out_specs=out_specs,
        out_shape=out_shape,
    )(*args)


def kernel(x_sub, x_hru, ei_ss, ei_hs, ei_sh, params):
    p = params
    src_ss, dst_ss = _pad_edges(ei_ss, E_SS_PAD, N_SUB)
    src_hs, dst_hs = _pad_edges(ei_hs, E_HS_PAD, N_HRU)

    seg_hs = _make_seg_sum(E_HS_PAD)
    seg_ss = _make_seg_sum(E_SS_PAD)
    counts = _make_counts((E_SS_PAD, E_HS_PAD))

    b2 = lambda v: v.reshape(1, -1)

    cnt_ss, cnt_hs = counts(dst_ss, dst_hs)
    ap = seg_hs(src_hs, dst_hs, x_hru)

    s0p = seg_ss(src_ss, dst_ss, x_sub)
    sub1, p1 = _layer(x_sub, s0p, ap, None, None,
                      p['Wr_0_ss'], p['Wl_0_hs'], p['Wr_0_hs'],
                      b2(p['bl_0_ss']), b2(p['bl_0_hs']), p['Wl_1_ss'], None,
                      use_mean=False, do_relu=True, final=False,
                      wlss=p['Wl_0_ss'])

    s1p = seg_ss(src_ss, dst_ss, p1)
    sub2, p2 = _layer(sub1, s1p, ap, cnt_ss, cnt_hs,
                      p['Wr_1_ss'], p['Wl_1_hs'], p['Wr_1_hs'],
                      b2(p['bl_1_ss']), b2(p['bl_1_hs']), p['Wl_2_ss'], None,
                      use_mean=True, do_relu=True, final=False)

    s2p = seg_ss(src_ss, dst_ss, p2)
    out = _layer(sub2, s2p, ap, cnt_ss, cnt_hs,
                 p['Wr_2_ss'], p['Wl_2_hs'], p['Wr_2_hs'],
                 b2(p['bl_2_ss']), b2(p['bl_2_hs']), p['W_fin'],
                 b2(p['b_fin']),
                 use_mean=True, do_relu=False, final=True)
    return out


# speedup vs baseline: 12.2369x; 1.0345x over previous
import functools

import jax
import jax.numpy as jnp
from jax import lax
from jax.experimental import pallas as pl
from jax.experimental.pallas import tpu as pltpu
from jax.experimental.pallas import tpu_sc as plsc

N_SUB = 10000
N_HRU = 50000
HID = 128
OUT = 16

NC = 2
NS = 16
NW = NC * NS
K = 56
CW = 16

N_PAD = 10240
ROWS_PT = N_PAD // NS
E_SS_PAD = NW * 92 * K
E_HS_PAD = NW * 172 * K

KC = 128
E_CNT_SS_PAD = NW * 44 * KC
E_CNT_HS_PAD = NW * 76 * KC


def _spans(total, step):
    out = []
    off = 0
    while off < total:
        out.append((off, min(step, total - off)))
        off += step
    return out


def _pad_edges(ei, e_pad, n_src, k):
    e = ei.shape[1]
    extra = jnp.arange(e_pad - e, dtype=jnp.int32)
    src = jnp.concatenate([ei[0].astype(jnp.int32), extra % n_src])
    dst = jnp.concatenate([ei[1].astype(jnp.int32),
                           N_SUB + extra % (N_PAD - N_SUB)])
    return src.reshape(e_pad // k, k), dst.reshape(e_pad // k, k)


_SC_PARAMS = pltpu.CompilerParams(use_tc_tiling_on_sc=False)
_SC_MESH = dict(core_axis_name="c", subcore_axis_name="s")


def _make_seg_sum(e_pad):
    c = e_pad // (NW * K)
    assert c % 4 == 0 and c >= 8

    scratch = [
        pltpu.VMEM((c, K), jnp.int32), pltpu.VMEM((c, K), jnp.int32),
    ] + [pltpu.VMEM((K, HID), jnp.float32) for _ in range(4)] + [
        pltpu.SemaphoreType.DMA for _ in range(8)
    ] + [pltpu.VMEM_SHARED((N_PAD, HID), jnp.float32)]

    def body(src_h, dst_h, tab_h, out_h,
             src_all, dst_all, r0, r1, r2, r3,
             g0, g1, g2, g3, s0, s1, s2, s3, acc):
        cid = lax.axis_index("c")
        sid = lax.axis_index("s")
        wid = sid * NC + cid
        cbase = wid * c
        rbase = sid * ROWS_PT

        rows = (r0, r1, r2, r3)
        gsem = (g0, g1, g2, g3)
        ssem = (s0, s1, s2, s3)
        rows0 = r0

        pltpu.sync_copy(src_h.at[pl.ds(cbase, c)], src_all)
        pltpu.sync_copy(dst_h.at[pl.ds(cbase, c)], dst_all)

        z16 = jnp.zeros((16,), jnp.float32)

        def zrow(r, _):
            for l in range(HID // 16):
                rows0[r, pl.ds(l * 16, 16)] = z16
            return 0
        lax.fori_loop(0, K, zrow, 0)

        for off, sz in _spans(ROWS_PT, K):
            pltpu.sync_copy(rows0.at[pl.ds(0, sz)],
                            acc.at[pl.ds(rbase + off, sz)])

        plsc.subcore_barrier()

        def g_start(j, b):
            pltpu.async_copy(tab_h.at[src_all.at[j]], rows[b], gsem[b])

        def g_wait(j, b):
            pltpu.make_async_copy(tab_h.at[src_all.at[j]], rows[b],
                                  gsem[b]).wait()

        def s_start(j, b):
            pltpu.async_copy(rows[b], acc.at[dst_all.at[j]], ssem[b],
                             add=True)

        def s_wait(j, b):
            pltpu.make_async_copy(rows[b], acc.at[dst_all.at[j]],
                                  ssem[b]).wait()

        g_start(0, 0)
        g_start(1, 1)
        g_wait(0, 0)
        s_start(0, 0)
        g_start(2, 2)
        g_wait(1, 1)
        s_start(1, 1)
        g_start(3, 3)

        def quad(g, _):
            j0 = 2 + 4 * g
            for t, b in enumerate((2, 3, 0, 1)):
                j = j0 + t
                g_wait(j, b)
                s_start(j, b)
                s_wait(j - 2, (b + 2) % 4)
                g_start(j + 2, (b + 2) % 4)
            return 0
        lax.fori_loop(0, (c - 4) // 4, quad, 0)

        g_wait(c - 2, 2)
        s_start(c - 2, 2)
        s_wait(c - 4, 0)
        g_wait(c - 1, 3)
        s_start(c - 1, 3)
        s_wait(c - 3, 1)
        s_wait(c - 2, 2)
        s_wait(c - 1, 3)

        plsc.subcore_barrier()

        pltpu.sync_copy(acc.at[pl.ds(rbase, ROWS_PT)],
                        out_h.at[cid, pl.ds(rbase, ROWS_PT)])

    return pl.kernel(
        body,
        out_type=jax.ShapeDtypeStruct((NC, N_PAD, HID), jnp.float32),
        mesh=plsc.VectorSubcoreMesh(**_SC_MESH),
        scratch_types=scratch,
        compiler_params=_SC_PARAMS)


def _make_counts(e_pads):
    chunks = [e // (NW * KC) for e in e_pads]
    assert all(c % 4 == 0 and c >= 8 for c in chunks)

    scratch = [pltpu.VMEM((c, KC), jnp.int32) for c in chunks] + [
        pltpu.VMEM((KC, CW), jnp.float32),
        pltpu.VMEM((KC, CW), jnp.float32),
        pltpu.SemaphoreType.DMA, pltpu.SemaphoreType.DMA,
        pltpu.SemaphoreType.DMA, pltpu.SemaphoreType.DMA,
    ] + [pltpu.VMEM_SHARED((N_PAD, CW), jnp.float32) for _ in e_pads]

    def body(*refs):
        n = len(e_pads)
        dst_hs = refs[:n]
        out_hs = refs[n:2 * n]
        dst_alls = refs[2 * n:3 * n]
        ones, zc = refs[3 * n:3 * n + 2]
        csem = refs[3 * n + 2:3 * n + 6]
        caccs = refs[3 * n + 6:]

        cid = lax.axis_index("c")
        sid = lax.axis_index("s")
        wid = sid * NC + cid
        rbase = sid * ROWS_PT

        z16 = jnp.zeros((16,), jnp.float32)
        onerow = jnp.where(lax.iota(jnp.int32, 16) == 0,
                           1.0, 0.0).astype(jnp.float32)

        def frow(r, _):
            ones[r] = onerow
            zc[r] = z16
            return 0
        lax.fori_loop(0, KC, frow, 0)

        for dst_h, dst_all, c in zip(dst_hs, dst_alls, chunks):
            pltpu.sync_copy(dst_h.at[pl.ds(wid * c, c)], dst_all)
        for cacc in caccs:
            for off, sz in _spans(ROWS_PT, KC):
                pltpu.sync_copy(zc.at[pl.ds(0, sz)],
                                cacc.at[pl.ds(rbase + off, sz)])
        plsc.subcore_barrier()

        for dst_all, cacc, c in zip(dst_alls, caccs, chunks):
            def c_start(j, b):
                pltpu.async_copy(ones, cacc.at[dst_all.at[j]], csem[b],
                                 add=True)

            def c_wait(j, b):
                pltpu.make_async_copy(ones, cacc.at[dst_all.at[j]],
                                      csem[b]).wait()

            for t in range(4):
                c_start(t, t)

            def quad(g, _):
                for t in range(4):
                    j = 4 * (g + 1) + t
                    c_wait(j - 4, t)
                    c_start(j, t)
                return 0
            lax.fori_loop(0, c // 4 - 1, quad, 0)
            for t in range(4):
                c_wait(c - 4 + t, t)

        plsc.subcore_barrier()
        for out_h, cacc in zip(out_hs, caccs):
            pltpu.sync_copy(cacc.at[pl.ds(rbase, ROWS_PT)],
                            out_h.at[cid, pl.ds(rbase, ROWS_PT)])

    out_type = tuple(jax.ShapeDtypeStruct((NC, N_PAD, CW), jnp.float32)
                     for _ in e_pads)
    return pl.kernel(body, out_type=out_type,
                     mesh=plsc.VectorSubcoreMesh(**_SC_MESH),
                     scratch_types=scratch,
                     compiler_params=_SC_PARAMS)


_BLK = 2000


def _dot(a, b):
    return jnp.dot(a, b, preferred_element_type=jnp.float32)


def _dot_hi(a, b):
    return jnp.dot(a, b, preferred_element_type=jnp.float32,
                   precision=lax.Precision.HIGHEST)


def _layer_body(use_mean, do_relu, final, proj_ss, *refs):
    refs = list(refs)
    x_ref, sp_ref, ap_ref = refs[:3]
    del refs[:3]
    if use_mean:
        css_ref, chs_ref = refs[:2]
        del refs[:2]
    else:
        css_ref = chs_ref = None
    if proj_ss:
        wlss_ref = refs.pop(0)
    else:
        wlss_ref = None
    wrss_ref, wlhs_ref, wrhs_ref, bss_ref, bhs_ref, wn_ref = refs[:6]
    del refs[:6]
    if final:
        bn_ref_or_none = refs.pop(0)
    else:
        bn_ref_or_none = None
    outs = refs

    x = x_ref[...]
    s = sp_ref[0] + sp_ref[1]
    a = ap_ref[0] + ap_ref[1]
    if use_mean:
        css = css_ref[0, :, 0:1] + css_ref[1, :, 0:1]
        chs = chs_ref[0, :, 0:1] + chs_ref[1, :, 0:1]
        s = s / jnp.maximum(css, 1.0)
        a = a / jnp.maximum(chs, 1.0)
    if proj_ss:
        s = _dot(s, wlss_ref[...])
    out_ss = s + bss_ref[...] + _dot(x, wrss_ref[...])
    out_hs = _dot(a, wlhs_ref[...]) + bhs_ref[...] + _dot(x, wrhs_ref[...])
    h = jnp.concatenate([out_ss, out_hs], axis=1)
    if do_relu:
        h = jnp.maximum(h, 0.0)
    if final:
        o = _dot(h, wn_ref[...]) + bn_ref_or_none[...]
        m = jnp.max(o, axis=1, keepdims=True)
        e = jnp.exp(o - m)
        outs[0][...] = e / jnp.sum(e, axis=1, keepdims=True)
    else:
        outs[0][...] = h
        outs[1][...] = _dot_hi(h, wn_ref[...])


def _layer(x, sp, ap, css, chs, wrss, wlhs, wrhs, bss, bhs, wn, bn,
           use_mean, do_relu, final, wlss=None):
    d = x.shape[1]
    full = lambda shape: pl.BlockSpec(shape, lambda i: tuple(0 for _ in shape))
    in_specs = [pl.BlockSpec((_BLK, d), lambda i: (i, 0)),
                pl.BlockSpec((NC, _BLK, HID), lambda i: (0, i, 0)),
                pl.BlockSpec((NC, _BLK, HID), lambda i: (0, i, 0))]
    args = [x, sp, ap]
    if use_mean:
        in_specs += [pl.BlockSpec((NC, _BLK, CW), lambda i: (0, i, 0)),
                     pl.BlockSpec((NC, _BLK, CW), lambda i: (0, i, 0))]
        args += [css, chs]
    if wlss is not None:
        in_specs.append(full(wlss.shape))
        args.append(wlss)
    in_specs += [full(wrss.shape), full(wlhs.shape), full(wrhs.shape),
                 full(bss.shape), full(bhs.shape), full(wn.shape)]
    args += [wrss, wlhs, wrhs, bss, bhs, wn]
    if final:
        in_specs.append(full(bn.shape))
        args.append(bn)
        out_specs = pl.BlockSpec((_BLK, OUT), lambda i: (i, 0))
        out_shape = jax.ShapeDtypeStruct((N_SUB, OUT), jnp.float32)
    else:
        out_specs = [pl.BlockSpec((_BLK, 2 * HID), lambda i: (i, 0)),
                     pl.BlockSpec((_BLK, HID), lambda i: (i, 0))]
        out_shape = [jax.ShapeDtypeStruct((N_SUB, 2 * HID), jnp.float32),
                     jax.ShapeDtypeStruct((N_SUB, HID), jnp.float32)]

    return pl.pallas_call(
        functools.partial(_layer_body, use_mean, do_relu, final,
                          wlss is not None),
        grid=(N_SUB // _BLK,),
        in_specs=in_specs,
        out_specs=out_specs,
        out_shape=out_shape,
    )(*args)


def kernel(x_sub, x_hru, ei_ss, ei_hs, ei_sh, params):
    p = params
    src_ss, dst_ss = _pad_edges(ei_ss, E_SS_PAD, N_SUB, K)
    src_hs, dst_hs = _pad_edges(ei_hs, E_HS_PAD, N_HRU, K)
    _, dst_cnt_ss = _pad_edges(ei_ss, E_CNT_SS_PAD, N_SUB, KC)
    _, dst_cnt_hs = _pad_edges(ei_hs, E_CNT_HS_PAD, N_HRU, KC)

    seg_hs = _make_seg_sum(E_HS_PAD)
    seg_ss = _make_seg_sum(E_SS_PAD)
    counts = _make_counts((E_CNT_SS_PAD, E_CNT_HS_PAD))

    b2 = lambda v: v.reshape(1, -1)

    cnt_ss, cnt_hs = counts(dst_cnt_ss, dst_cnt_hs)
    ap = seg_hs(src_hs, dst_hs, x_hru)

    s0p = seg_ss(src_ss, dst_ss, x_sub)
    sub1, p1 = _layer(x_sub, s0p, ap, None, None,
                      p['Wr_0_ss'], p['Wl_0_hs'], p['Wr_0_hs'],
                      b2(p['bl_0_ss']), b2(p['bl_0_hs']), p['Wl_1_ss'], None,
                      use_mean=False, do_relu=True, final=False,
                      wlss=p['Wl_0_ss'])

    s1p = seg_ss(src_ss, dst_ss, p1)
    sub2, p2 = _layer(sub1, s1p, ap, cnt_ss, cnt_hs,
                      p['Wr_1_ss'], p['Wl_1_hs'], p['Wr_1_hs'],
                      b2(p['bl_1_ss']), b2(p['bl_1_hs']), p['Wl_2_ss'], None,
                      use_mean=True, do_relu=True, final=False)

    s2p = seg_ss(src_ss, dst_ss, p2)
    out = _layer(sub2, s2p, ap, cnt_ss, cnt_hs,
                 p['Wr_2_ss'], p['Wl_2_hs'], p['Wr_2_hs'],
                 b2(p['bl_2_ss']), b2(p['bl_2_hs']), p['W_fin'],
                 b2(p['b_fin']),
                 use_mean=True, do_relu=False, final=True)
    return out


# speedup vs baseline: 12.4465x; 1.0171x over previous
import functools

import jax
import jax.numpy as jnp
from jax import lax
from jax.experimental import pallas as pl
from jax.experimental.pallas import tpu as pltpu
from jax.experimental.pallas import tpu_sc as plsc

N_SUB = 10000
N_HRU = 50000
HID = 128
OUT = 16

NC = 2
NS = 16
NW = NC * NS
K = 56
CW = 16

N_PAD = 10240
ROWS_PT = N_PAD // NS
E_SS_PAD = NW * 92 * K
E_HS_PAD = NW * 172 * K


def _spans(total, step):
    out = []
    off = 0
    while off < total:
        out.append((off, min(step, total - off)))
        off += step
    return out


def _pad_edges(ei, e_pad, n_src, k):
    e = ei.shape[1]
    extra = jnp.arange(e_pad - e, dtype=jnp.int32)
    src = jnp.concatenate([ei[0].astype(jnp.int32), extra % n_src])
    dst = jnp.concatenate([ei[1].astype(jnp.int32),
                           N_SUB + extra % (N_PAD - N_SUB)])
    return src.reshape(e_pad // k, k), dst.reshape(e_pad // k, k)


_SC_PARAMS = pltpu.CompilerParams(use_tc_tiling_on_sc=False)
_SC_MESH = dict(core_axis_name="c", subcore_axis_name="s")


def _make_seg_sum(e_pad, with_count=False):
    c = e_pad // (NW * K)
    assert c % 4 == 0 and c >= 8

    out_type = [jax.ShapeDtypeStruct((NC, N_PAD, HID), jnp.float32)]
    scratch = [
        pltpu.VMEM((c, K), jnp.int32), pltpu.VMEM((c, K), jnp.int32),
    ] + [pltpu.VMEM((K, HID), jnp.float32) for _ in range(4)] + [
        pltpu.SemaphoreType.DMA for _ in range(8)
    ] + [pltpu.VMEM_SHARED((N_PAD, HID), jnp.float32)]
    if with_count:
        out_type.append(jax.ShapeDtypeStruct((NC, N_PAD), jnp.float32))
        scratch += [
            pltpu.VMEM((64,), jnp.float32),
            pltpu.VMEM((64,), jnp.float32),
            pltpu.SemaphoreType.DMA, pltpu.SemaphoreType.DMA,
            pltpu.SemaphoreType.DMA, pltpu.SemaphoreType.DMA,
            pltpu.VMEM_SHARED((N_PAD,), jnp.float32),
        ]

    def body(*refs):
        if with_count:
            (src_h, dst_h, tab_h, out_h, cnt_h,
             src_all, dst_all, r0, r1, r2, r3,
             g0, g1, g2, g3, s0, s1, s2, s3, acc,
             ones1, zb1, c0, c1, c2, c3, cacc) = refs
            csem = (c0, c1, c2, c3)
        else:
            (src_h, dst_h, tab_h, out_h,
             src_all, dst_all, r0, r1, r2, r3,
             g0, g1, g2, g3, s0, s1, s2, s3, acc) = refs
            cnt_h = ones1 = zb1 = csem = cacc = None
        cid = lax.axis_index("c")
        sid = lax.axis_index("s")
        wid = sid * NC + cid
        cbase = wid * c
        rbase = sid * ROWS_PT

        rows = (r0, r1, r2, r3)
        gsem = (g0, g1, g2, g3)
        ssem = (s0, s1, s2, s3)
        rows0 = r0

        pltpu.sync_copy(src_h.at[pl.ds(cbase, c)], src_all)
        pltpu.sync_copy(dst_h.at[pl.ds(cbase, c)], dst_all)

        z16 = jnp.zeros((16,), jnp.float32)

        def zrow(r, _):
            for l in range(HID // 16):
                rows0[r, pl.ds(l * 16, 16)] = z16
            return 0
        lax.fori_loop(0, K, zrow, 0)

        for off, sz in _spans(ROWS_PT, K):
            pltpu.sync_copy(rows0.at[pl.ds(0, sz)],
                            acc.at[pl.ds(rbase + off, sz)])

        if with_count:
            o16 = jnp.full((16,), 1.0, jnp.float32)

            def c_fill(r, _):
                ones1[pl.ds(r * 16, 16)] = o16
                return 0
            lax.fori_loop(0, 4, c_fill, 0)

            def z_fill(r, _):
                zb1[pl.ds(r * 16, 16)] = z16
                return 0
            lax.fori_loop(0, 4, z_fill, 0)
            for off in range(0, ROWS_PT, 64):
                pltpu.sync_copy(zb1, cacc.at[pl.ds(rbase + off, 64)])

        plsc.subcore_barrier()

        def g_start(j, b):
            pltpu.async_copy(tab_h.at[src_all.at[j]], rows[b], gsem[b])

        def g_wait(j, b):
            pltpu.make_async_copy(tab_h.at[src_all.at[j]], rows[b],
                                  gsem[b]).wait()

        def s_start(j, b):
            pltpu.async_copy(rows[b], acc.at[dst_all.at[j]], ssem[b],
                             add=True)
            if with_count:
                pltpu.async_copy(ones1.at[pl.ds(0, K)],
                                 cacc.at[dst_all.at[j]], csem[b], add=True)

        def s_wait(j, b):
            pltpu.make_async_copy(rows[b], acc.at[dst_all.at[j]],
                                  ssem[b]).wait()
            if with_count:
                pltpu.make_async_copy(ones1.at[pl.ds(0, K)],
                                      cacc.at[dst_all.at[j]],
                                      csem[b]).wait()

        g_start(0, 0)
        g_start(1, 1)
        g_wait(0, 0)
        s_start(0, 0)
        g_start(2, 2)
        g_wait(1, 1)
        s_start(1, 1)
        g_start(3, 3)

        def quad(g, _):
            j0 = 2 + 4 * g
            for t, b in enumerate((2, 3, 0, 1)):
                j = j0 + t
                g_wait(j, b)
                s_start(j, b)
                s_wait(j - 2, (b + 2) % 4)
                g_start(j + 2, (b + 2) % 4)
            return 0
        lax.fori_loop(0, (c - 4) // 4, quad, 0)

        g_wait(c - 2, 2)
        s_start(c - 2, 2)
        s_wait(c - 4, 0)
        g_wait(c - 1, 3)
        s_start(c - 1, 3)
        s_wait(c - 3, 1)
        s_wait(c - 2, 2)
        s_wait(c - 1, 3)

        plsc.subcore_barrier()

        pltpu.sync_copy(acc.at[pl.ds(rbase, ROWS_PT)],
                        out_h.at[cid, pl.ds(rbase, ROWS_PT)])
        if with_count:
            pltpu.sync_copy(cacc.at[pl.ds(rbase, ROWS_PT)],
                            cnt_h.at[cid, pl.ds(rbase, ROWS_PT)])

    return pl.kernel(
        body,
        out_type=tuple(out_type) if with_count else out_type[0],
        mesh=plsc.VectorSubcoreMesh(**_SC_MESH),
        scratch_types=scratch,
        compiler_params=_SC_PARAMS)


_BLK = 2000


def _dot(a, b):
    return jnp.dot(a, b, preferred_element_type=jnp.float32)


def _dot_hi(a, b):
    return jnp.dot(a, b, preferred_element_type=jnp.float32,
                   precision=lax.Precision.HIGHEST)


def _layer_body(use_mean, do_relu, final, proj_ss, *refs):
    refs = list(refs)
    x_ref, sp_ref, ap_ref = refs[:3]
    del refs[:3]
    if use_mean:
        css_ref, chs_ref = refs[:2]
        del refs[:2]
    else:
        css_ref = chs_ref = None
    if proj_ss:
        wlss_ref = refs.pop(0)
    else:
        wlss_ref = None
    wrss_ref, wlhs_ref, wrhs_ref, bss_ref, bhs_ref, wn_ref = refs[:6]
    del refs[:6]
    if final:
        bn_ref_or_none = refs.pop(0)
    else:
        bn_ref_or_none = None
    outs = refs

    x = x_ref[...]
    s = sp_ref[0] + sp_ref[1]
    a = ap_ref[0] + ap_ref[1]
    if use_mean:
        css = (css_ref[:, 0] + css_ref[:, 1])[:, None]
        chs = (chs_ref[:, 0] + chs_ref[:, 1])[:, None]
        s = s / jnp.maximum(css, 1.0)
        a = a / jnp.maximum(chs, 1.0)
    if proj_ss:
        s = _dot(s, wlss_ref[...])
    out_ss = s + bss_ref[...] + _dot(x, wrss_ref[...])
    out_hs = _dot(a, wlhs_ref[...]) + bhs_ref[...] + _dot(x, wrhs_ref[...])
    h = jnp.concatenate([out_ss, out_hs], axis=1)
    if do_relu:
        h = jnp.maximum(h, 0.0)
    if final:
        o = _dot(h, wn_ref[...]) + bn_ref_or_none[...]
        m = jnp.max(o, axis=1, keepdims=True)
        e = jnp.exp(o - m)
        outs[0][...] = e / jnp.sum(e, axis=1, keepdims=True)
    else:
        outs[0][...] = h
        outs[1][...] = _dot_hi(h, wn_ref[...])


def _layer(x, sp, ap, css, chs, wrss, wlhs, wrhs, bss, bhs, wn, bn,
           use_mean, do_relu, final, wlss=None):
    d = x.shape[1]
    full = lambda shape: pl.BlockSpec(shape, lambda i: tuple(0 for _ in shape))
    in_specs = [pl.BlockSpec((_BLK, d), lambda i: (i, 0)),
                pl.BlockSpec((NC, _BLK, HID), lambda i: (0, i, 0)),
                pl.BlockSpec((NC, _BLK, HID), lambda i: (0, i, 0))]
    args = [x, sp, ap]
    if use_mean:
        in_specs += [pl.BlockSpec((_BLK, NC), lambda i: (i, 0)),
                     pl.BlockSpec((_BLK, NC), lambda i: (i, 0))]
        args += [css, chs]
    if wlss is not None:
        in_specs.append(full(wlss.shape))
        args.append(wlss)
    in_specs += [full(wrss.shape), full(wlhs.shape), full(wrhs.shape),
                 full(bss.shape), full(bhs.shape), full(wn.shape)]
    args += [wrss, wlhs, wrhs, bss, bhs, wn]
    if final:
        in_specs.append(full(bn.shape))
        args.append(bn)
        out_specs = pl.BlockSpec((_BLK, OUT), lambda i: (i, 0))
        out_shape = jax.ShapeDtypeStruct((N_SUB, OUT), jnp.float32)
    else:
        out_specs = [pl.BlockSpec((_BLK, 2 * HID), lambda i: (i, 0)),
                     pl.BlockSpec((_BLK, HID), lambda i: (i, 0))]
        out_shape = [jax.ShapeDtypeStruct((N_SUB, 2 * HID), jnp.float32),
                     jax.ShapeDtypeStruct((N_SUB, HID), jnp.float32)]

    return pl.pallas_call(
        functools.partial(_layer_body, use_mean, do_relu, final,
                          wlss is not None),
        grid=(N_SUB // _BLK,),
        in_specs=in_specs,
        out_specs=out_specs,
        out_shape=out_shape,
    )(*args)


def kernel(x_sub, x_hru, ei_ss, ei_hs, ei_sh, params):
    p = params
    src_ss, dst_ss = _pad_edges(ei_ss, E_SS_PAD, N_SUB, K)
    src_hs, dst_hs = _pad_edges(ei_hs, E_HS_PAD, N_HRU, K)

    seg_hs_cnt = _make_seg_sum(E_HS_PAD, with_count=True)
    seg_ss_cnt = _make_seg_sum(E_SS_PAD, with_count=True)
    seg_ss = _make_seg_sum(E_SS_PAD)

    b2 = lambda v: v.reshape(1, -1)

    ap, cnt_hs = seg_hs_cnt(src_hs, dst_hs, x_hru)
    cnt_hs = cnt_hs.T

    s0p, cnt_ss = seg_ss_cnt(src_ss, dst_ss, x_sub)
    cnt_ss = cnt_ss.T
    sub1, p1 = _layer(x_sub, s0p, ap, None, None,
                      p['Wr_0_ss'], p['Wl_0_hs'], p['Wr_0_hs'],
                      b2(p['bl_0_ss']), b2(p['bl_0_hs']), p['Wl_1_ss'], None,
                      use_mean=False, do_relu=True, final=False,
                      wlss=p['Wl_0_ss'])

    s1p = seg_ss(src_ss, dst_ss, p1)
    sub2, p2 = _layer(sub1, s1p, ap, cnt_ss, cnt_hs,
                      p['Wr_1_ss'], p['Wl_1_hs'], p['Wr_1_hs'],
                      b2(p['bl_1_ss']), b2(p['bl_1_hs']), p['Wl_2_ss'], None,
                      use_mean=True, do_relu=True, final=False)

    s2p = seg_ss(src_ss, dst_ss, p2)
    out = _layer(sub2, s2p, ap, cnt_ss, cnt_hs,
                 p['Wr_2_ss'], p['Wl_2_hs'], p['Wr_2_hs'],
                 b2(p['bl_2_ss']), b2(p['bl_2_hs']), p['W_fin'],
                 b2(p['b_fin']),
                 use_mean=True, do_relu=False, final=True)
    return out
